# baseline (speedup 1.0000x reference)
import functools

import jax
import jax.numpy as jnp
from jax import lax
from jax.experimental import pallas as pl
from jax.experimental.pallas import tpu as pltpu
from jax.experimental.pallas import tpu_sc as plsc

_VOCAB = 1_000_000
_MAXLEN = 200
_EMBED = 32
_BATCH = 4096

_L = 16
_NC = 2
_NS = 16
_NW = _NC * _NS
_BB = _BATCH // _NW
_TB = 8
_NTB = _MAXLEN // _TB
_PAIRS = (_NTB - 1) // 2

_mesh = plsc.VectorSubcoreMesh(core_axis_name="c", subcore_axis_name="s")


@functools.partial(
    pl.kernel,
    out_type=jax.ShapeDtypeStruct((_MAXLEN, _EMBED // 8, _NW, 8, 128), jnp.float32),
    mesh=_mesh,
    scratch_types=[
        pltpu.VMEM((_BB, _TB), jnp.int32),
        pltpu.VMEM((_BB, _TB), jnp.int32),
        pltpu.VMEM((_TB, _BB), jnp.int32),
        pltpu.VMEM((_TB, _BB), jnp.int32),
        pltpu.VMEM((_TB * _BB, _EMBED), jnp.float32),
        pltpu.VMEM((_TB * _BB, _EMBED), jnp.float32),
        pltpu.VMEM((_TB, _EMBED // 8, 1, 8, _BB), jnp.float32),
        pltpu.VMEM((_MAXLEN, _EMBED), jnp.float32),
        pltpu.SemaphoreType.DMA,
        pltpu.SemaphoreType.DMA,
    ],
    compiler_params=pltpu.CompilerParams(use_tc_tiling_on_sc=False,
                                         needs_layout_passes=False),
)
def _tok_pos_embed(x_hbm, tok_hbm, pos_hbm, out_hbm,
                   xblk0, xblk1, idx0, idx1, rows0, rows1,
                   obuf_v, pos_v, sem0, sem1):
    w = lax.axis_index("s") * _NC + lax.axis_index("c")
    pltpu.sync_copy(pos_hbm, pos_v)

    lane = jnp.arange(_L, dtype=jnp.int32)
    et_lo = lane >> 3
    es_lo = lane & 7
    et_hi = (lane + _L) >> 3
    es_hi = (lane + _L) & 7
    zero = jnp.zeros((_L,), dtype=jnp.int32)

    xblks = (xblk0, xblk1)
    idxs = (idx0, idx1)
    rows = (rows0, rows1)
    sems = (sem0, sem1)

    def stage(buf, tt):
        xblk, idx_v = xblks[buf], idxs[buf]
        pltpu.sync_copy(
            x_hbm.at[pl.ds(w * _BB, _BB), pl.ds(tt * _TB, _TB)], xblk)
        for s in range(_TB):
            scol = jnp.full((_L,), s, dtype=jnp.int32)
            for lg in range(_BB // _L):
                v = plsc.load_gather(xblk, [lane + (lg * _L), scol])
                idx_v[s, pl.ds(lg * _L, _L)] = v

    def fire(buf):
        for s in range(_TB):
            pltpu.make_async_copy(
                tok_hbm.at[idxs[buf].at[s]],
                rows[buf].at[pl.ds(s * _BB, _BB)],
                sems[buf],
            ).start()

    def drain(buf):
        for s in range(_TB):
            pltpu.make_async_copy(
                tok_hbm.at[idxs[buf].at[s]],
                rows[buf].at[pl.ds(s * _BB, _BB)],
                sems[buf],
            ).wait()

    def scatter_out(buf, tt):
        rows_v = rows[buf]
        for s in range(_TB):
            t = tt * _TB + s
            tvec = jnp.full((_L,), t, dtype=jnp.int32)

            def ebody(e, acc):
                evec = jnp.full((_L,), e, dtype=jnp.int32)
                pv = plsc.load_gather(pos_v, [tvec, evec])
                for k in range(_BB // _L):
                    jvec = lane + (s * _BB + k * _L)
                    v = plsc.load_gather(rows_v, [jvec, evec])
                    obuf_v[s, e >> 3, 0, e & 7, pl.ds(k * _L, _L)] = v + pv
                return acc

            lax.fori_loop(0, _EMBED, ebody, 0, unroll=2)

        pltpu.sync_copy(
            obuf_v,
            out_hbm.at[pl.ds(tt * _TB, _TB), slice(None), pl.ds(w, 1)])

    stage(0, 0)
    fire(0)

    def double_step(tt2, carry):
        tt_e = tt2 * 2
        stage(1, tt_e + 1)
        fire(1)
        drain(0)
        scatter_out(0, tt_e)
        stage(0, tt_e + 2)
        fire(0)
        drain(1)
        scatter_out(1, tt_e + 1)
        return carry

    lax.fori_loop(0, _PAIRS, double_step, 0)

    drain(0)
    scatter_out(0, _NTB - 1)


def kernel(x, token_table, pos_table):
    out5 = _tok_pos_embed(x.astype(jnp.int32), token_table, pos_table)
    return out5.transpose(2, 4, 0, 1, 3).reshape(_BATCH, _MAXLEN, _EMBED)


# speedup vs baseline: 1.1577x; 1.1577x over previous
import functools

import jax
import jax.numpy as jnp
from jax import lax
from jax.experimental import pallas as pl
from jax.experimental.pallas import tpu as pltpu
from jax.experimental.pallas import tpu_sc as plsc

_VOCAB = 1_000_000
_MAXLEN = 200
_EMBED = 32
_BATCH = 4096

_L = 16
_NC = 2
_NS = 16
_NW = _NC * _NS
_BB = _BATCH // _NW
_TB = 8
_NTB = _MAXLEN // _TB
_PAIRS = (_NTB - 1) // 2

_mesh = plsc.VectorSubcoreMesh(core_axis_name="c", subcore_axis_name="s")


@functools.partial(
    pl.kernel,
    out_type=jax.ShapeDtypeStruct((_MAXLEN, _EMBED // 8, _NW, 8, 128), jnp.float32),
    mesh=_mesh,
    scratch_types=[
        pltpu.VMEM((_BB, _TB), jnp.int32),
        pltpu.VMEM((_BB, _TB), jnp.int32),
        pltpu.VMEM((_TB, _BB), jnp.int32),
        pltpu.VMEM((_TB, _BB), jnp.int32),
        pltpu.VMEM((_TB * _BB, _EMBED), jnp.float32),
        pltpu.VMEM((_TB * _BB, _EMBED), jnp.float32),
        pltpu.VMEM((_TB, _EMBED // 8, 1, 8, _BB), jnp.float32),
        pltpu.VMEM((_MAXLEN, _EMBED), jnp.float32),
        pltpu.SemaphoreType.DMA,
        pltpu.SemaphoreType.DMA,
    ],
    compiler_params=pltpu.CompilerParams(use_tc_tiling_on_sc=False,
                                         needs_layout_passes=False),
)
def _tok_pos_embed(x_hbm, tok_hbm, pos_hbm, out_hbm,
                   xblk0, xblk1, idx0, idx1, rows0, rows1,
                   obuf_v, pos_v, sem0, sem1):
    w = lax.axis_index("s") * _NC + lax.axis_index("c")
    pltpu.sync_copy(pos_hbm, pos_v)

    lane = jnp.arange(_L, dtype=jnp.int32)
    et_lo = lane >> 3
    es_lo = lane & 7
    et_hi = (lane + _L) >> 3
    es_hi = (lane + _L) & 7
    zero = jnp.zeros((_L,), dtype=jnp.int32)

    xblks = (xblk0, xblk1)
    idxs = (idx0, idx1)
    rows = (rows0, rows1)
    sems = (sem0, sem1)

    def stage(buf, tt):
        xblk, idx_v = xblks[buf], idxs[buf]
        pltpu.sync_copy(
            x_hbm.at[pl.ds(w * _BB, _BB), pl.ds(tt * _TB, _TB)], xblk)
        for s in range(_TB):
            scol = jnp.full((_L,), s, dtype=jnp.int32)
            for lg in range(_BB // _L):
                v = plsc.load_gather(xblk, [lane + (lg * _L), scol])
                idx_v[s, pl.ds(lg * _L, _L)] = v

    def fire(buf):
        for s in range(_TB):
            pltpu.make_async_copy(
                tok_hbm.at[idxs[buf].at[s]],
                rows[buf].at[pl.ds(s * _BB, _BB)],
                sems[buf],
            ).start()

    def drain(buf):
        for s in range(_TB):
            pltpu.make_async_copy(
                tok_hbm.at[idxs[buf].at[s]],
                rows[buf].at[pl.ds(s * _BB, _BB)],
                sems[buf],
            ).wait()

    def scatter_out(buf, tt):
        rows_v = rows[buf]
        for s in range(_TB):
            t = tt * _TB + s
            pos_lo = pos_v[t, pl.ds(0, _L)]
            pos_hi = pos_v[t, pl.ds(_L, _L)]
            flat0 = (lane << 7) + (s * 4096)

            def scat(l, flat):
                j = s * _BB + l
                v0 = rows_v[j, pl.ds(0, _L)] + pos_lo
                v1 = rows_v[j, pl.ds(_L, _L)] + pos_hi
                plsc.store_scatter(obuf_v, [zero, zero, zero, zero, flat], v0)
                plsc.store_scatter(obuf_v, [zero, zero, zero, zero, flat + 2048], v1)
                return flat + 1

            lax.fori_loop(0, _BB, scat, flat0, unroll=8)

        pltpu.sync_copy(
            obuf_v,
            out_hbm.at[pl.ds(tt * _TB, _TB), slice(None), pl.ds(w, 1)])

    stage(0, 0)
    fire(0)

    def double_step(tt2, carry):
        tt_e = tt2 * 2
        stage(1, tt_e + 1)
        fire(1)
        drain(0)
        scatter_out(0, tt_e)
        stage(0, tt_e + 2)
        fire(0)
        drain(1)
        scatter_out(1, tt_e + 1)
        return carry

    lax.fori_loop(0, _PAIRS, double_step, 0)

    drain(0)
    scatter_out(0, _NTB - 1)


def kernel(x, token_table, pos_table):
    out5 = _tok_pos_embed(x.astype(jnp.int32), token_table, pos_table)
    return out5.transpose(2, 4, 0, 1, 3).reshape(_BATCH, _MAXLEN, _EMBED)


# speedup vs baseline: 1.7193x; 1.4851x over previous
import functools

import jax
import jax.numpy as jnp
from jax import lax
from jax.experimental import pallas as pl
from jax.experimental.pallas import tpu as pltpu
from jax.experimental.pallas import tpu_sc as plsc

_VOCAB = 1_000_000
_MAXLEN = 200
_EMBED = 32
_BATCH = 4096

_L = 16
_NC = 2
_NS = 16
_NW = _NC * _NS
_BB = _BATCH // _NW
_TB = 8
_NTB = _MAXLEN // _TB
_PAIRS = (_NTB - 1) // 2

_mesh = plsc.VectorSubcoreMesh(core_axis_name="c", subcore_axis_name="s")


@functools.partial(
    pl.kernel,
    out_type=jax.ShapeDtypeStruct((_MAXLEN, _EMBED // 8, _NW, 8, 128), jnp.float32),
    mesh=_mesh,
    scratch_types=[
        pltpu.VMEM((_BB, _TB), jnp.int32),
        pltpu.VMEM((_BB, _TB), jnp.int32),
        pltpu.VMEM((_TB, _BB), jnp.int32),
        pltpu.VMEM((_TB, _BB), jnp.int32),
        pltpu.VMEM((_TB * _BB, _EMBED), jnp.float32),
        pltpu.VMEM((_TB * _BB, _EMBED), jnp.float32),
        pltpu.VMEM((_TB, _EMBED // 8, 1, 8, _BB), jnp.float32),
        pltpu.VMEM((_MAXLEN, _EMBED), jnp.float32),
        pltpu.SemaphoreType.DMA,
        pltpu.SemaphoreType.DMA,
    ],
    compiler_params=pltpu.CompilerParams(use_tc_tiling_on_sc=False,
                                         needs_layout_passes=False),
)
def _tok_pos_embed(x_hbm, tok_hbm, pos_hbm, out_hbm,
                   xblk0, xblk1, idx0, idx1, rows0, rows1,
                   obuf_v, pos_v, sem0, sem1):
    w = lax.axis_index("s") * _NC + lax.axis_index("c")
    pltpu.sync_copy(pos_hbm, pos_v)

    lane = jnp.arange(_L, dtype=jnp.int32)
    et_lo = lane >> 3
    es_lo = lane & 7
    et_hi = (lane + _L) >> 3
    es_hi = (lane + _L) & 7
    zero = jnp.zeros((_L,), dtype=jnp.int32)

    xblks = (xblk0, xblk1)
    idxs = (idx0, idx1)
    rows = (rows0, rows1)
    sems = (sem0, sem1)

    def stage(buf, tt):
        xblk, idx_v = xblks[buf], idxs[buf]
        pltpu.sync_copy(
            x_hbm.at[pl.ds(w * _BB, _BB), pl.ds(tt * _TB, _TB)], xblk)
        for s in range(_TB):
            scol = jnp.full((_L,), s, dtype=jnp.int32)
            for lg in range(_BB // _L):
                v = plsc.load_gather(xblk, [lane + (lg * _L), scol])
                idx_v[s, pl.ds(lg * _L, _L)] = v

    def fire(buf):
        for s in range(_TB):
            pltpu.make_async_copy(
                tok_hbm.at[idxs[buf].at[s]],
                rows[buf].at[pl.ds(s * _BB, _BB)],
                sems[buf],
            ).start()

    def drain(buf):
        for s in range(_TB):
            pltpu.make_async_copy(
                tok_hbm.at[idxs[buf].at[s]],
                rows[buf].at[pl.ds(s * _BB, _BB)],
                sems[buf],
            ).wait()

    def scatter_out(buf, tt):
        rows_v = rows[buf]
        lane128 = lane << 7
        for s in range(_TB):
            t = tt * _TB + s
            pos_lo = pos_v[t, pl.ds(0, _L)]
            pos_hi = pos_v[t, pl.ds(_L, _L)]

            def lblock(lb, acc):
                j0 = s * _BB + lb * _L
                jsplat = jnp.full((_L,), j0, dtype=jnp.int32)
                dbase = lane128 + (s * 4096 + lb * _L)

                def diag(d, acc2):
                    rot = (lane + d) & 15
                    jvec = jsplat + rot
                    v0 = plsc.load_gather(rows_v, [jvec, lane]) + pos_lo
                    v1 = plsc.load_gather(rows_v, [jvec, lane + _L]) + pos_hi
                    off = dbase + rot
                    plsc.store_scatter(obuf_v, [zero, zero, zero, zero, off], v0)
                    plsc.store_scatter(obuf_v, [zero, zero, zero, zero, off + 2048], v1)
                    return acc2

                lax.fori_loop(0, _L, diag, 0, unroll=4)
                return acc

            lax.fori_loop(0, _BB // _L, lblock, 0)

        pltpu.sync_copy(
            obuf_v,
            out_hbm.at[pl.ds(tt * _TB, _TB), slice(None), pl.ds(w, 1)])

    stage(0, 0)
    fire(0)

    def double_step(tt2, carry):
        tt_e = tt2 * 2
        stage(1, tt_e + 1)
        fire(1)
        drain(0)
        scatter_out(0, tt_e)
        stage(0, tt_e + 2)
        fire(0)
        drain(1)
        scatter_out(1, tt_e + 1)
        return carry

    lax.fori_loop(0, _PAIRS, double_step, 0)

    drain(0)
    scatter_out(0, _NTB - 1)


def kernel(x, token_table, pos_table):
    out5 = _tok_pos_embed(x.astype(jnp.int32), token_table, pos_table)
    return out5.transpose(2, 4, 0, 1, 3).reshape(_BATCH, _MAXLEN, _EMBED)
